# baseline (speedup 1.0000x reference)
import functools

import jax
import jax.numpy as jnp
from jax import lax
from jax.experimental import pallas as pl
from jax.experimental.pallas import tpu as pltpu
from jax.experimental.pallas import tpu_sc as plsc

NC = 2
NS = 16
NW = NC * NS

B = 16384
D = 32
V = 1_000_000
BPW = B // NW
CHUNK = 128
NCHUNK = BPW // CHUNK
L = 16

STRIP = 2048
GRID = -(-V // STRIP)
PR = GRID * 512

_mesh = plsc.VectorSubcoreMesh(
    core_axis_name="c", subcore_axis_name="s", num_cores=NC, num_subcores=NS
)


def _repack_body(zt_ref, out_ref):
    eye = jnp.eye(D, dtype=jnp.float32)
    y = jax.lax.dot_general(
        zt_ref[...], eye, (((0,), (1,)), ((), ())),
        preferred_element_type=jnp.float32)
    out_ref[...] = jnp.concatenate(
        [y[q * 512:(q + 1) * 512, :] for q in range(4)], axis=1)


_repack = pl.pallas_call(
    _repack_body,
    grid=(GRID,),
    in_specs=[pl.BlockSpec((D, STRIP), lambda i: (0, i))],
    out_specs=pl.BlockSpec((512, 128), lambda i: (i, 0)),
    out_shape=jax.ShapeDtypeStruct((PR, 128), jnp.float32),
)


@functools.partial(
    pl.kernel,
    mesh=_mesh,
    out_type=jax.ShapeDtypeStruct((B,), jnp.float32),
    compiler_params=pltpu.CompilerParams(needs_layout_passes=False),
    scratch_types=[
        pltpu.VMEM((2, NCHUNK, CHUNK), jnp.int32),
        pltpu.VMEM((1, CHUNK), jnp.int32),
        pltpu.VMEM((1, CHUNK), jnp.int32),
        pltpu.VMEM((CHUNK, 128), jnp.float32),
        pltpu.VMEM((CHUNK, 128), jnp.float32),
        pltpu.VMEM((BPW,), jnp.float32),
        pltpu.SemaphoreType.DMA,
    ],
)
def _sc_dot_decoder(z128_hbm, h_hbm, t_hbm, out_hbm,
                    ids, ridx_h, ridx_t, gbuf_h, gbuf_t, out_v, sem):
    wid = lax.axis_index("s") * NC + lax.axis_index("c")
    base = wid * BPW

    for s, src in enumerate((h_hbm, t_hbm)):
        for c in range(NCHUNK):
            pltpu.sync_copy(src.at[pl.ds(base + c * CHUNK, CHUNK)],
                            ids.at[s, c])

    lane = jnp.arange(L, dtype=jnp.int32)

    def row_of(v):
        return ((v >> 11) << 9) | (v & 511)

    def off_of(v):
        return ((v >> 9) & 3) * D

    def chunk_body(c, carry):
        for k in range(CHUNK // L):
            sl = pl.ds(k * L, L)
            ridx_h[0, sl] = row_of(ids[0, c, sl])
            ridx_t[0, sl] = row_of(ids[1, c, sl])
        cp1 = pltpu.async_copy(z128_hbm.at[ridx_h.at[0]], gbuf_h, sem)
        cp2 = pltpu.async_copy(z128_hbm.at[ridx_t.at[0]], gbuf_t, sem)
        cp1.wait()
        cp2.wait()

        for g in range(CHUNK // L):
            sl = pl.ds(g * L, L)
            rows = g * L + lane
            offh = off_of(ids[0, c, sl])
            offt = off_of(ids[1, c, sl])
            acc = None
            for d in range(D):
                hv = plsc.load_gather(gbuf_h, [rows, offh + d])
                tv = plsc.load_gather(gbuf_t, [rows, offt + d])
                prod = hv * tv
                acc = prod if acc is None else acc + prod
            out_v[pl.ds(c * CHUNK + g * L, L)] = acc
        return carry

    lax.fori_loop(0, NCHUNK, chunk_body, 0)

    pltpu.sync_copy(out_v, out_hbm.at[pl.ds(base, BPW)])


def kernel(z, triples):
    z128 = _repack(z.T)
    h = triples[:, 0]
    t = triples[:, 2]
    return _sc_dot_decoder(z128, h, t)


# speedup vs baseline: 1.5510x; 1.5510x over previous
import functools

import jax
import jax.numpy as jnp
from jax import lax
from jax.experimental import pallas as pl
from jax.experimental.pallas import tpu as pltpu
from jax.experimental.pallas import tpu_sc as plsc

NC = 2
NS = 16
NW = NC * NS

B = 16384
D = 32
V = 1_000_000
BPW = B // NW
CHUNK = 128
NCHUNK = BPW // CHUNK
L = 16

STRIP = 8192
QR = STRIP // 4
GRID = -(-V // STRIP)
PR = GRID * QR

_mesh = plsc.VectorSubcoreMesh(
    core_axis_name="c", subcore_axis_name="s", num_cores=NC, num_subcores=NS
)


def _repack_body(zt_ref, out_ref):
    y = zt_ref[...].T
    out_ref[...] = jnp.concatenate(
        [y[q * QR:(q + 1) * QR, :] for q in range(4)], axis=1)


_repack = pl.pallas_call(
    _repack_body,
    grid=(GRID,),
    in_specs=[pl.BlockSpec((D, STRIP), lambda i: (0, i))],
    out_specs=pl.BlockSpec((QR, 128), lambda i: (i, 0)),
    out_shape=jax.ShapeDtypeStruct((PR, 128), jnp.float32),
)


@functools.partial(
    pl.kernel,
    mesh=_mesh,
    out_type=jax.ShapeDtypeStruct((B,), jnp.float32),
    compiler_params=pltpu.CompilerParams(needs_layout_passes=False),
    scratch_types=[
        pltpu.VMEM((2, NCHUNK, CHUNK), jnp.int32),
        pltpu.VMEM((1, CHUNK), jnp.int32),
        pltpu.VMEM((1, CHUNK), jnp.int32),
        pltpu.VMEM((CHUNK, 128), jnp.float32),
        pltpu.VMEM((CHUNK, 128), jnp.float32),
        pltpu.VMEM((BPW,), jnp.float32),
        pltpu.SemaphoreType.DMA,
    ],
)
def _sc_dot_decoder(z128_hbm, h_hbm, t_hbm, out_hbm,
                    ids, ridx_h, ridx_t, gbuf_h, gbuf_t, out_v, sem):
    wid = lax.axis_index("s") * NC + lax.axis_index("c")
    base = wid * BPW

    for s, src in enumerate((h_hbm, t_hbm)):
        for c in range(NCHUNK):
            pltpu.sync_copy(src.at[pl.ds(base + c * CHUNK, CHUNK)],
                            ids.at[s, c])

    lane = jnp.arange(L, dtype=jnp.int32)

    def row_of(v):
        return ((v >> 13) << 11) | (v & (QR - 1))

    def off_of(v):
        return ((v >> 11) & 3) * D

    def chunk_body(c, carry):
        for k in range(CHUNK // L):
            sl = pl.ds(k * L, L)
            ridx_h[0, sl] = row_of(ids[0, c, sl])
            ridx_t[0, sl] = row_of(ids[1, c, sl])
        cp1 = pltpu.async_copy(z128_hbm.at[ridx_h.at[0]], gbuf_h, sem)
        cp2 = pltpu.async_copy(z128_hbm.at[ridx_t.at[0]], gbuf_t, sem)
        cp1.wait()
        cp2.wait()

        for g in range(CHUNK // L):
            sl = pl.ds(g * L, L)
            rows = g * L + lane
            offh = off_of(ids[0, c, sl])
            offt = off_of(ids[1, c, sl])
            acc = None
            for d in range(D):
                hv = plsc.load_gather(gbuf_h, [rows, offh + d])
                tv = plsc.load_gather(gbuf_t, [rows, offt + d])
                prod = hv * tv
                acc = prod if acc is None else acc + prod
            out_v[pl.ds(c * CHUNK + g * L, L)] = acc
        return carry

    lax.fori_loop(0, NCHUNK, chunk_body, 0)

    pltpu.sync_copy(out_v, out_hbm.at[pl.ds(base, BPW)])


def kernel(z, triples):
    z128 = _repack(z.T)
    h = triples[:, 0]
    t = triples[:, 2]
    return _sc_dot_decoder(z128, h, t)


# speedup vs baseline: 1.5777x; 1.0172x over previous
import functools

import jax
import jax.numpy as jnp
from jax import lax
from jax.experimental import pallas as pl
from jax.experimental.pallas import tpu as pltpu
from jax.experimental.pallas import tpu_sc as plsc

NC = 2
NS = 16
NW = NC * NS

B = 16384
D = 32
V = 1_000_000
BPW = B // NW
CHUNK = 128
NCHUNK = BPW // CHUNK
L = 16

STRIP = 32768
QR = STRIP // 4
GRID = -(-V // STRIP)
PR = GRID * QR

_mesh = plsc.VectorSubcoreMesh(
    core_axis_name="c", subcore_axis_name="s", num_cores=NC, num_subcores=NS
)


def _repack_body(zt_ref, out_ref):
    y = zt_ref[...].T
    out_ref[...] = jnp.concatenate(
        [y[q * QR:(q + 1) * QR, :] for q in range(4)], axis=1)


_repack = pl.pallas_call(
    _repack_body,
    grid=(GRID,),
    in_specs=[pl.BlockSpec((D, STRIP), lambda i: (0, i))],
    out_specs=pl.BlockSpec((QR, 128), lambda i: (i, 0)),
    out_shape=jax.ShapeDtypeStruct((PR, 128), jnp.float32),
)


@functools.partial(
    pl.kernel,
    mesh=_mesh,
    out_type=jax.ShapeDtypeStruct((B,), jnp.float32),
    compiler_params=pltpu.CompilerParams(needs_layout_passes=False),
    scratch_types=[
        pltpu.VMEM((2, NCHUNK, CHUNK), jnp.int32),
        pltpu.VMEM((1, CHUNK), jnp.int32),
        pltpu.VMEM((1, CHUNK), jnp.int32),
        pltpu.VMEM((CHUNK, 128), jnp.float32),
        pltpu.VMEM((CHUNK, 128), jnp.float32),
        pltpu.VMEM((BPW,), jnp.float32),
        pltpu.SemaphoreType.DMA,
    ],
)
def _sc_dot_decoder(z128_hbm, h_hbm, t_hbm, out_hbm,
                    ids, ridx_h, ridx_t, gbuf_h, gbuf_t, out_v, sem):
    wid = lax.axis_index("s") * NC + lax.axis_index("c")
    base = wid * BPW

    for s, src in enumerate((h_hbm, t_hbm)):
        for c in range(NCHUNK):
            pltpu.sync_copy(src.at[pl.ds(base + c * CHUNK, CHUNK)],
                            ids.at[s, c])

    lane = jnp.arange(L, dtype=jnp.int32)

    def row_of(v):
        return ((v >> 15) << 13) | (v & (QR - 1))

    def off_of(v):
        return ((v >> 13) & 3) * D

    def chunk_body(c, carry):
        for k in range(CHUNK // L):
            sl = pl.ds(k * L, L)
            ridx_h[0, sl] = row_of(ids[0, c, sl])
            ridx_t[0, sl] = row_of(ids[1, c, sl])
        cp1 = pltpu.async_copy(z128_hbm.at[ridx_h.at[0]], gbuf_h, sem)
        cp2 = pltpu.async_copy(z128_hbm.at[ridx_t.at[0]], gbuf_t, sem)
        cp1.wait()
        cp2.wait()

        for g in range(CHUNK // L):
            sl = pl.ds(g * L, L)
            rows = g * L + lane
            offh = off_of(ids[0, c, sl])
            offt = off_of(ids[1, c, sl])
            acc = None
            for d in range(D):
                hv = plsc.load_gather(gbuf_h, [rows, offh + d])
                tv = plsc.load_gather(gbuf_t, [rows, offt + d])
                prod = hv * tv
                acc = prod if acc is None else acc + prod
            out_v[pl.ds(c * CHUNK + g * L, L)] = acc
        return carry

    lax.fori_loop(0, NCHUNK, chunk_body, 0)

    pltpu.sync_copy(out_v, out_hbm.at[pl.ds(base, BPW)])


def kernel(z, triples):
    z128 = _repack(z.T)
    h = triples[:, 0]
    t = triples[:, 2]
    return _sc_dot_decoder(z128, h, t)


# speedup vs baseline: 1.5800x; 1.0015x over previous
import functools

import jax
import jax.numpy as jnp
from jax import lax
from jax.experimental import pallas as pl
from jax.experimental.pallas import tpu as pltpu
from jax.experimental.pallas import tpu_sc as plsc

NC = 2
NS = 16
NW = NC * NS

B = 16384
D = 32
V = 1_000_000
BPW = B // NW
CHUNK = 128
NCHUNK = BPW // CHUNK
L = 16

STRIP = 32768
QR = STRIP // 4
GRID = -(-V // STRIP)
PR = GRID * QR

_mesh = plsc.VectorSubcoreMesh(
    core_axis_name="c", subcore_axis_name="s", num_cores=NC, num_subcores=NS
)


def _repack_body(zt_ref, out_ref):
    eye = jnp.eye(D, dtype=jnp.float32)
    y = jax.lax.dot_general(
        zt_ref[...], eye, (((0,), (1,)), ((), ())),
        preferred_element_type=jnp.float32)
    out_ref[...] = jnp.concatenate(
        [y[q * QR:(q + 1) * QR, :] for q in range(4)], axis=1)


_repack = pl.pallas_call(
    _repack_body,
    grid=(GRID,),
    in_specs=[pl.BlockSpec((D, STRIP), lambda i: (0, i))],
    out_specs=pl.BlockSpec((QR, 128), lambda i: (i, 0)),
    out_shape=jax.ShapeDtypeStruct((PR, 128), jnp.float32),
)


@functools.partial(
    pl.kernel,
    mesh=_mesh,
    out_type=jax.ShapeDtypeStruct((B,), jnp.float32),
    compiler_params=pltpu.CompilerParams(needs_layout_passes=False),
    scratch_types=[
        pltpu.VMEM((2, NCHUNK, CHUNK), jnp.int32),
        pltpu.VMEM((1, CHUNK), jnp.int32),
        pltpu.VMEM((1, CHUNK), jnp.int32),
        pltpu.VMEM((CHUNK, 128), jnp.float32),
        pltpu.VMEM((CHUNK, 128), jnp.float32),
        pltpu.VMEM((BPW,), jnp.float32),
        pltpu.SemaphoreType.DMA,
    ],
)
def _sc_dot_decoder(z128_hbm, h_hbm, t_hbm, out_hbm,
                    ids, ridx_h, ridx_t, gbuf_h, gbuf_t, out_v, sem):
    wid = lax.axis_index("s") * NC + lax.axis_index("c")
    base = wid * BPW

    for s, src in enumerate((h_hbm, t_hbm)):
        for c in range(NCHUNK):
            pltpu.sync_copy(src.at[pl.ds(base + c * CHUNK, CHUNK)],
                            ids.at[s, c])

    lane = jnp.arange(L, dtype=jnp.int32)

    def row_of(v):
        return ((v >> 15) << 13) | (v & (QR - 1))

    def off_of(v):
        return ((v >> 13) & 3) * D

    def chunk_body(c, carry):
        for k in range(CHUNK // L):
            sl = pl.ds(k * L, L)
            ridx_h[0, sl] = row_of(ids[0, c, sl])
            ridx_t[0, sl] = row_of(ids[1, c, sl])
        cp1 = pltpu.async_copy(z128_hbm.at[ridx_h.at[0]], gbuf_h, sem)
        cp2 = pltpu.async_copy(z128_hbm.at[ridx_t.at[0]], gbuf_t, sem)
        cp1.wait()
        cp2.wait()

        for g in range(CHUNK // L):
            sl = pl.ds(g * L, L)
            rows = g * L + lane
            offh = off_of(ids[0, c, sl])
            offt = off_of(ids[1, c, sl])
            acc = None
            for d in range(D):
                hv = plsc.load_gather(gbuf_h, [rows, offh + d])
                tv = plsc.load_gather(gbuf_t, [rows, offt + d])
                prod = hv * tv
                acc = prod if acc is None else acc + prod
            out_v[pl.ds(c * CHUNK + g * L, L)] = acc
        return carry

    lax.fori_loop(0, NCHUNK, chunk_body, 0)

    pltpu.sync_copy(out_v, out_hbm.at[pl.ds(base, BPW)])


def kernel(z, triples):
    z128 = _repack(z.T)
    h = triples[:, 0]
    t = triples[:, 2]
    return _sc_dot_decoder(z128, h, t)
